# Initial kernel scaffold; baseline (speedup 1.0000x reference)
import jax
import jax.numpy as jnp
from jax.experimental import pallas as pl


def kernel(tokens, emb_table):
    raise NotImplementedError("write your pallas kernel here")


# speedup vs baseline: 1.0578x; 1.0578x over previous
import functools

import jax
import jax.numpy as jnp
from jax import lax
from jax.experimental import pallas as pl
from jax.experimental.pallas import tpu as pltpu
from jax.experimental.pallas import tpu_sc as plsc

NUM_EMB = 1_000_000
DIM = 32
BATCH = 16384
HIST = 50

NC, NS = 2, 16
NW = NC * NS
TOTAL = BATCH * HIST
PER_W = TOTAL // NW
CHUNK = 128
NCHUNK = PER_W // CHUNK
NBUF = 4


def _sc_gather(tokens_flat, emb_table):
    mesh = plsc.VectorSubcoreMesh(
        core_axis_name="c", subcore_axis_name="s", num_cores=NC, num_subcores=NS
    )

    @functools.partial(
        pl.kernel,
        mesh=mesh,
        out_type=jax.ShapeDtypeStruct((TOTAL, DIM), jnp.float32),
        scratch_types=[
            pltpu.VMEM((NBUF, CHUNK), jnp.int32),
            pltpu.VMEM((NBUF, CHUNK, DIM), jnp.float32),
            pltpu.SemaphoreType.DMA((NBUF,)),
        ],
        compiler_params=pltpu.CompilerParams(use_tc_tiling_on_sc=False),
    )
    def k(idx_hbm, table_hbm, out_hbm, idx_v, rows_v, sems):
        wid = lax.axis_index("s") * NC + lax.axis_index("c")
        base = wid * PER_W

        def fire(g, b):
            off = base + g * CHUNK
            pltpu.sync_copy(idx_hbm.at[pl.ds(off, CHUNK)], idx_v.at[b])
            pltpu.async_copy(table_hbm.at[idx_v.at[b]], rows_v.at[b], sems.at[b])

        def drain(g, b):
            pltpu.make_async_copy(
                table_hbm.at[idx_v.at[b]], rows_v.at[b], sems.at[b]
            ).wait()
            off = base + g * CHUNK
            pltpu.sync_copy(rows_v.at[b], out_hbm.at[pl.ds(off, CHUNK)])

        for b in range(NBUF):
            fire(b, b)

        def body(i, _):
            for b in range(NBUF):
                g = i * NBUF + b
                drain(g, b)
                nxt = g + NBUF

                @pl.when(nxt < NCHUNK)
                def _():
                    fire(nxt, b)

            return ()

        lax.fori_loop(0, NCHUNK // NBUF, body, (), unroll=False)

    return k(tokens_flat, emb_table)


def kernel(tokens, emb_table):
    tokens_flat = tokens.reshape(TOTAL).astype(jnp.int32)
    out = _sc_gather(tokens_flat, emb_table)
    return out.reshape(BATCH, HIST, DIM)


# speedup vs baseline: 1.1133x; 1.0525x over previous
import functools

import jax
import jax.numpy as jnp
from jax import lax
from jax.experimental import pallas as pl
from jax.experimental.pallas import tpu as pltpu
from jax.experimental.pallas import tpu_sc as plsc

NUM_EMB = 1_000_000
DIM = 32
BATCH = 16384
HIST = 50

NC, NS = 2, 16
NW = NC * NS
TOTAL = BATCH * HIST
PER_W = TOTAL // NW
CHUNK = 128
NCHUNK = PER_W // CHUNK
G = 8
NG = NCHUNK // G


def _sc_gather(tokens_grp, emb_table):
    mesh = plsc.VectorSubcoreMesh(
        core_axis_name="c", subcore_axis_name="s", num_cores=NC, num_subcores=NS
    )

    @functools.partial(
        pl.kernel,
        mesh=mesh,
        out_type=jax.ShapeDtypeStruct((TOTAL, DIM), jnp.float32),
        scratch_types=[
            pltpu.VMEM((NCHUNK, CHUNK), jnp.int32),
            pltpu.VMEM((2, G, CHUNK, DIM), jnp.float32),
            pltpu.SemaphoreType.DMA((2,)),
            pltpu.SemaphoreType.DMA((2,)),
        ],
        compiler_params=pltpu.CompilerParams(use_tc_tiling_on_sc=False),
    )
    def k(idx_hbm, table_hbm, out_hbm, idx_v, rows_v, gsem, ssem):
        wid = lax.axis_index("s") * NC + lax.axis_index("c")
        base = wid * PER_W

        pltpu.sync_copy(idx_hbm.at[wid], idx_v)

        def fire_gathers(i, p):
            for b in range(G):
                pltpu.async_copy(
                    table_hbm.at[idx_v.at[i * G + b]],
                    rows_v.at[p, b],
                    gsem.at[p],
                )

        def wait_gathers(i, p):
            for b in range(G):
                pltpu.make_async_copy(
                    table_hbm.at[idx_v.at[i * G + b]],
                    rows_v.at[p, b],
                    gsem.at[p],
                ).wait()

        def fire_stores(i, p):
            for b in range(G):
                off = base + (i * G + b) * CHUNK
                pltpu.async_copy(
                    rows_v.at[p, b],
                    out_hbm.at[pl.ds(off, CHUNK)],
                    ssem.at[p],
                )

        def wait_stores(i, p):
            for b in range(G):
                off = base + (i * G + b) * CHUNK
                pltpu.make_async_copy(
                    rows_v.at[p, b],
                    out_hbm.at[pl.ds(off, CHUNK)],
                    ssem.at[p],
                ).wait()

        fire_gathers(0, 0)

        def body(i, _):
            p = lax.rem(i, 2)
            q = 1 - p

            @pl.when(i >= 1)
            def _():
                wait_stores(i - 1, q)

            @pl.when(i + 1 < NG)
            def _():
                fire_gathers(i + 1, q)

            wait_gathers(i, p)
            fire_stores(i, p)
            return ()

        lax.fori_loop(0, NG, body, (), unroll=False)
        wait_stores(NG - 1, (NG - 1) % 2)

    return k(tokens_grp, emb_table)


def kernel(tokens, emb_table):
    tokens_grp = tokens.reshape(NW, NCHUNK, CHUNK).astype(jnp.int32)
    out = _sc_gather(tokens_grp, emb_table)
    return out.reshape(BATCH, HIST, DIM)
